# baseline (speedup 1.0000x reference)
import math
import functools

import jax
import jax.numpy as jnp
from jax import lax
from jax.experimental import pallas as pl
from jax.experimental.pallas import tpu as pltpu

N_ATOMS_C = 10000
K_NB = 32
M_PAD = 10112
TILE = 128
N_TILES = M_PAD // TILE

_TET_TA = 0.6081734479693927 * math.pi
_TET_IDT = 1.0 / (12.0 * math.pi / 180.0 + 1e-10)
_C20 = 0.25 * math.sqrt(5.0 / math.pi)
_A21 = -0.5 * math.sqrt(15.0 / (2.0 * math.pi))
_A22 = 0.25 * math.sqrt(15.0 / (2.0 * math.pi))
_Q2S = 4.0 * math.pi / 5.0


def _acos(x):
    return jnp.arctan2(jnp.sqrt((1.0 - x) * (1.0 + x)), x)


def _bf(x):
    return x.astype(jnp.bfloat16).astype(jnp.float32)


def _bf_bits(x):
    u = jax.lax.bitcast_convert_type(x.astype(jnp.float32), jnp.uint32)
    bias = jnp.uint32(0x7FFF) + ((u >> 16) & jnp.uint32(1))
    u = (u + bias) & jnp.uint32(0xFFFF0000)
    return jax.lax.bitcast_convert_type(u, jnp.float32)


def _dense_body(cell_ref, cinv_ref, cen_ref, cnt_ref, nx_ref, ny_ref, nz_ref,
                cn_ref, tet_ref, q2_ref):
    K = K_NB
    f32 = jnp.float32
    krow = lax.broadcasted_iota(jnp.int32, (K, TILE), 0)
    cnt = cnt_ref[0:1, :]
    valid = krow < cnt
    vm = valid.astype(f32)

    cx = cen_ref[0:1, :]
    cy = cen_ref[1:2, :]
    cz = cen_ref[2:3, :]
    px = nx_ref[...]
    py = ny_ref[...]
    pz = nz_ref[...]
    px = jnp.where(valid, px, cx)
    py = jnp.where(valid, py, cy)
    pz = jnp.where(valid, pz, cz)
    dx = _bf(px - cx)
    dy = _bf(py - cy)
    dz = _bf(pz - cz)

    ci00 = cinv_ref[0, 0]; ci01 = cinv_ref[0, 1]; ci02 = cinv_ref[0, 2]
    ci10 = cinv_ref[1, 0]; ci11 = cinv_ref[1, 1]; ci12 = cinv_ref[1, 2]
    ci20 = cinv_ref[2, 0]; ci21 = cinv_ref[2, 1]; ci22 = cinv_ref[2, 2]
    c00 = cell_ref[0, 0]; c01 = cell_ref[0, 1]; c02 = cell_ref[0, 2]
    c10 = cell_ref[1, 0]; c11 = cell_ref[1, 1]; c12 = cell_ref[1, 2]
    c20 = cell_ref[2, 0]; c21 = cell_ref[2, 1]; c22 = cell_ref[2, 2]

    fx = dx * ci00 + dy * ci10 + dz * ci20
    fy = dx * ci01 + dy * ci11 + dz * ci21
    fz = dx * ci02 + dy * ci12 + dz * ci22
    fx = _bf(fx - jnp.round(fx))
    fy = _bf(fy - jnp.round(fy))
    fz = _bf(fz - jnp.round(fz))
    rx = fx * c00 + fy * c10 + fz * c20
    ry = fx * c01 + fy * c11 + fz * c21
    rz = fx * c02 + fy * c12 + fz * c22
    rx = (cx + rx) - cx
    ry = (cy + ry) - cy
    rz = (cz + rz) - cz

    d = jnp.sqrt(rx * rx + ry * ry + rz * rz + 1e-10)
    inv = 1.0 / (d + 1e-10)
    vx = rx * inv
    vy = ry * inv
    vz = rz * inv
    cos_t = jnp.clip(vz, -1.0 + 1e-7, 1.0 - 1e-7)
    th = _acos(cos_t)
    ph = jnp.arctan2(vy, vx)
    vx = jnp.where(valid, vx, 0.0)
    vy = jnp.where(valid, vy, 0.0)
    vz = jnp.where(valid, vz, 0.0)
    th = jnp.where(valid, th, 0.0)
    ph = jnp.where(valid, ph, 0.0)

    cnf = jnp.sum(vm, axis=0, keepdims=True)
    cn_ref[...] = cnf

    ct = jnp.cos(th)
    st = jnp.sin(th)
    cp = jnp.cos(ph)
    sp = jnp.sin(ph)
    nb = jnp.maximum(cnf, 1.0)
    inb = 1.0 / nb
    y20 = _C20 * (3.0 * ct * ct - 1.0)
    q20 = jnp.sum(y20 * vm, axis=0, keepdims=True) * inb
    a21 = _A21 * st * ct
    re21 = jnp.sum(a21 * cp * vm, axis=0, keepdims=True) * inb
    im21 = jnp.sum(a21 * sp * vm, axis=0, keepdims=True) * inb
    a22 = _A22 * st * st
    c2p = cp * cp - sp * sp
    s2p = 2.0 * sp * cp
    re22 = jnp.sum(a22 * c2p * vm, axis=0, keepdims=True) * inb
    im22 = jnp.sum(a22 * s2p * vm, axis=0, keepdims=True) * inb
    s = q20 * q20 + 2.0 * (re21 * re21 + im21 * im21) + 2.0 * (re22 * re22 + im22 * im22)
    q2_ref[...] = jnp.sqrt(_Q2S * s + 1e-12)

    gacc = jnp.zeros((K, TILE), f32)
    vxb = _bf(vx)
    vyb = _bf(vy)
    vzb = _bf(vz)
    for k in range(K):
        bx = vxb[k:k + 1, :]
        by = vyb[k:k + 1, :]
        bz = vzb[k:k + 1, :]
        vmk = vm[k:k + 1, :]
        dots = vxb * bx + vyb * by + vzb * bz
        dots = jnp.clip(dots, -1.0 + 1e-7, 1.0 - 1e-7)
        ang = _acos(dots)
        t = (ang - _TET_TA) * _TET_IDT
        ex = jnp.maximum(-0.5 * t * t, -50.0)
        g = jnp.exp(ex)
        fm = vm * vmk * (krow != k).astype(f32)
        gacc = gacc + g * fm
    gs = jnp.sum(gacc, axis=0, keepdims=True)
    npairs = cnf * (cnf - 1.0)
    tet_ref[...] = gs / jnp.maximum(npairs, 1.0)


def _dense_call(cell, cinv, cen_t, cnt, nx, ny, nz):
    out_shape = [jax.ShapeDtypeStruct((1, M_PAD), jnp.float32)] * 3
    grid = (N_TILES,)
    smem = functools.partial(pl.BlockSpec, memory_space=pltpu.SMEM)
    return pl.pallas_call(
        _dense_body,
        grid=grid,
        in_specs=[
            smem((3, 3), lambda i: (0, 0)),
            smem((3, 3), lambda i: (0, 0)),
            pl.BlockSpec((3, TILE), lambda i: (0, i)),
            pl.BlockSpec((1, TILE), lambda i: (0, i)),
            pl.BlockSpec((K_NB, TILE), lambda i: (0, i)),
            pl.BlockSpec((K_NB, TILE), lambda i: (0, i)),
            pl.BlockSpec((K_NB, TILE), lambda i: (0, i)),
        ],
        out_specs=[
            pl.BlockSpec((1, TILE), lambda i: (0, i)),
            pl.BlockSpec((1, TILE), lambda i: (0, i)),
            pl.BlockSpec((1, TILE), lambda i: (0, i)),
        ],
        out_shape=out_shape,
        compiler_params=pltpu.CompilerParams(
            dimension_semantics=("arbitrary",),
        ),
    )(cell, cinv, cen_t, cnt, nx, ny, nz)


def _inv3(m):
    a, b, c = m[0, 0], m[0, 1], m[0, 2]
    d, e, f = m[1, 0], m[1, 1], m[1, 2]
    g, h, i = m[2, 0], m[2, 1], m[2, 2]
    A = e * i - f * h
    B = f * g - d * i
    C = d * h - e * g
    det = a * A + b * B + c * C
    adj = jnp.stack([
        jnp.stack([A, c * h - b * i, b * f - c * e]),
        jnp.stack([B, a * i - c * g, c * d - a * f]),
        jnp.stack([C, b * g - a * h, a * e - b * d]),
    ])
    return adj / det


def _build_neighbors_xla(positions, edge_index):
    src = edge_index[0]
    dst = edge_index[1]
    E = src.shape[0]
    order = jnp.argsort(src)
    src_s = src[order]
    dst_s = dst[order]
    first = jnp.searchsorted(src_s, src_s, side='left')
    rank = (jnp.arange(E, dtype=src.dtype) - first).astype(jnp.int32)
    keep = rank < K_NB
    flat = jnp.where(keep, rank * M_PAD + src_s.astype(jnp.int32), M_PAD * K_NB)
    g = positions[dst_s]
    zero = jnp.zeros((M_PAD * K_NB + 1,), jnp.float32)
    nx = zero.at[flat].set(g[:, 0])[:M_PAD * K_NB].reshape(K_NB, M_PAD)
    ny = zero.at[flat].set(g[:, 1])[:M_PAD * K_NB].reshape(K_NB, M_PAD)
    nz = zero.at[flat].set(g[:, 2])[:M_PAD * K_NB].reshape(K_NB, M_PAD)
    cnt = jnp.zeros((M_PAD,), jnp.int32).at[src].add(1)
    return nx, ny, nz, cnt


def kernel(positions, cell, edge_index, atom_indices):
    nx, ny, nz, cnt = _build_neighbors_xla(positions, edge_index)
    cinv = _inv3(cell)
    cell = _bf_bits(cell)
    cinv = _bf_bits(cinv)
    cen_t = jnp.zeros((3, M_PAD), jnp.float32).at[:, :N_ATOMS_C].set(positions.T)
    cn, tet, q2 = _dense_call(cell, cinv, cen_t, cnt.reshape(1, M_PAD), nx, ny, nz)
    n = atom_indices.shape[0]
    return (cn[0, :n], tet[0, :n], q2[0, :n])


# speedup vs baseline: 1.6483x; 1.6483x over previous
import math
import functools

import jax
import jax.numpy as jnp
from jax import lax
from jax.experimental import pallas as pl
from jax.experimental.pallas import tpu as pltpu
from jax.experimental.pallas import tpu_sc as plsc

N_ATOMS_C = 10000
K_NB = 32
M_PAD = 10112
TILE = 128
N_TILES = M_PAD // TILE

_TET_TA = 0.6081734479693927 * math.pi
_TET_IDT = 1.0 / (12.0 * math.pi / 180.0 + 1e-10)
_C20 = 0.25 * math.sqrt(5.0 / math.pi)
_A21 = -0.5 * math.sqrt(15.0 / (2.0 * math.pi))
_A22 = 0.25 * math.sqrt(15.0 / (2.0 * math.pi))
_Q2S = 4.0 * math.pi / 5.0


def _acos(x):
    return jnp.arctan2(jnp.sqrt((1.0 - x) * (1.0 + x)), x)


def _bf(x):
    return x.astype(jnp.bfloat16).astype(jnp.float32)


def _bf_bits(x):
    u = jax.lax.bitcast_convert_type(x.astype(jnp.float32), jnp.uint32)
    bias = jnp.uint32(0x7FFF) + ((u >> 16) & jnp.uint32(1))
    u = (u + bias) & jnp.uint32(0xFFFF0000)
    return jax.lax.bitcast_convert_type(u, jnp.float32)


def _dense_body(cell_ref, cinv_ref, cen_ref, cnt_ref, nx_ref, ny_ref, nz_ref,
                cn_ref, tet_ref, q2_ref):
    K = K_NB
    f32 = jnp.float32
    krow = lax.broadcasted_iota(jnp.int32, (K, TILE), 0)
    cnt = cnt_ref[0:1, :]
    valid = krow < cnt
    vm = valid.astype(f32)

    cx = cen_ref[0:1, :]
    cy = cen_ref[1:2, :]
    cz = cen_ref[2:3, :]
    px = nx_ref[...]
    py = ny_ref[...]
    pz = nz_ref[...]
    px = jnp.where(valid, px, cx)
    py = jnp.where(valid, py, cy)
    pz = jnp.where(valid, pz, cz)
    dx = _bf(px - cx)
    dy = _bf(py - cy)
    dz = _bf(pz - cz)

    ci00 = cinv_ref[0, 0]; ci01 = cinv_ref[0, 1]; ci02 = cinv_ref[0, 2]
    ci10 = cinv_ref[1, 0]; ci11 = cinv_ref[1, 1]; ci12 = cinv_ref[1, 2]
    ci20 = cinv_ref[2, 0]; ci21 = cinv_ref[2, 1]; ci22 = cinv_ref[2, 2]
    c00 = cell_ref[0, 0]; c01 = cell_ref[0, 1]; c02 = cell_ref[0, 2]
    c10 = cell_ref[1, 0]; c11 = cell_ref[1, 1]; c12 = cell_ref[1, 2]
    c20 = cell_ref[2, 0]; c21 = cell_ref[2, 1]; c22 = cell_ref[2, 2]

    fx = dx * ci00 + dy * ci10 + dz * ci20
    fy = dx * ci01 + dy * ci11 + dz * ci21
    fz = dx * ci02 + dy * ci12 + dz * ci22
    fx = _bf(fx - jnp.round(fx))
    fy = _bf(fy - jnp.round(fy))
    fz = _bf(fz - jnp.round(fz))
    rx = fx * c00 + fy * c10 + fz * c20
    ry = fx * c01 + fy * c11 + fz * c21
    rz = fx * c02 + fy * c12 + fz * c22
    rx = (cx + rx) - cx
    ry = (cy + ry) - cy
    rz = (cz + rz) - cz

    d = jnp.sqrt(rx * rx + ry * ry + rz * rz + 1e-10)
    inv = 1.0 / (d + 1e-10)
    vx = rx * inv
    vy = ry * inv
    vz = rz * inv
    cos_t = jnp.clip(vz, -1.0 + 1e-7, 1.0 - 1e-7)
    th = _acos(cos_t)
    ph = jnp.arctan2(vy, vx)
    vx = jnp.where(valid, vx, 0.0)
    vy = jnp.where(valid, vy, 0.0)
    vz = jnp.where(valid, vz, 0.0)
    th = jnp.where(valid, th, 0.0)
    ph = jnp.where(valid, ph, 0.0)

    cnf = jnp.sum(vm, axis=0, keepdims=True)
    cn_ref[...] = cnf

    ct = jnp.cos(th)
    st = jnp.sin(th)
    cp = jnp.cos(ph)
    sp = jnp.sin(ph)
    nb = jnp.maximum(cnf, 1.0)
    inb = 1.0 / nb
    y20 = _C20 * (3.0 * ct * ct - 1.0)
    q20 = jnp.sum(y20 * vm, axis=0, keepdims=True) * inb
    a21 = _A21 * st * ct
    re21 = jnp.sum(a21 * cp * vm, axis=0, keepdims=True) * inb
    im21 = jnp.sum(a21 * sp * vm, axis=0, keepdims=True) * inb
    a22 = _A22 * st * st
    c2p = cp * cp - sp * sp
    s2p = 2.0 * sp * cp
    re22 = jnp.sum(a22 * c2p * vm, axis=0, keepdims=True) * inb
    im22 = jnp.sum(a22 * s2p * vm, axis=0, keepdims=True) * inb
    s = q20 * q20 + 2.0 * (re21 * re21 + im21 * im21) + 2.0 * (re22 * re22 + im22 * im22)
    q2_ref[...] = jnp.sqrt(_Q2S * s + 1e-12)

    gacc = jnp.zeros((K, TILE), f32)
    vxb = _bf(vx)
    vyb = _bf(vy)
    vzb = _bf(vz)
    for k in range(K):
        bx = vxb[k:k + 1, :]
        by = vyb[k:k + 1, :]
        bz = vzb[k:k + 1, :]
        vmk = vm[k:k + 1, :]
        dots = vxb * bx + vyb * by + vzb * bz
        dots = jnp.clip(dots, -1.0 + 1e-7, 1.0 - 1e-7)
        ang = _acos(dots)
        t = (ang - _TET_TA) * _TET_IDT
        ex = jnp.maximum(-0.5 * t * t, -50.0)
        g = jnp.exp(ex)
        fm = vm * vmk * (krow != k).astype(f32)
        gacc = gacc + g * fm
    gs = jnp.sum(gacc, axis=0, keepdims=True)
    npairs = cnf * (cnf - 1.0)
    tet_ref[...] = gs / jnp.maximum(npairs, 1.0)


def _dense_call(cell, cinv, cen_t, cnt, nx, ny, nz):
    out_shape = [jax.ShapeDtypeStruct((1, M_PAD), jnp.float32)] * 3
    grid = (N_TILES,)
    smem = functools.partial(pl.BlockSpec, memory_space=pltpu.SMEM)
    return pl.pallas_call(
        _dense_body,
        grid=grid,
        in_specs=[
            smem((3, 3), lambda i: (0, 0)),
            smem((3, 3), lambda i: (0, 0)),
            pl.BlockSpec((3, TILE), lambda i: (0, i)),
            pl.BlockSpec((1, TILE), lambda i: (0, i)),
            pl.BlockSpec((K_NB, TILE), lambda i: (0, i)),
            pl.BlockSpec((K_NB, TILE), lambda i: (0, i)),
            pl.BlockSpec((K_NB, TILE), lambda i: (0, i)),
        ],
        out_specs=[
            pl.BlockSpec((1, TILE), lambda i: (0, i)),
            pl.BlockSpec((1, TILE), lambda i: (0, i)),
            pl.BlockSpec((1, TILE), lambda i: (0, i)),
        ],
        out_shape=out_shape,
        compiler_params=pltpu.CompilerParams(
            dimension_semantics=("arbitrary",),
        ),
    )(cell, cinv, cen_t, cnt, nx, ny, nz)


def _inv3(m):
    a, b, c = m[0, 0], m[0, 1], m[0, 2]
    d, e, f = m[1, 0], m[1, 1], m[1, 2]
    g, h, i = m[2, 0], m[2, 1], m[2, 2]
    A = e * i - f * h
    B = f * g - d * i
    C = d * h - e * g
    det = a * A + b * B + c * C
    adj = jnp.stack([
        jnp.stack([A, c * h - b * i, b * f - c * e]),
        jnp.stack([B, a * i - c * g, c * d - a * f]),
        jnp.stack([C, b * g - a * h, a * e - b * d]),
    ])
    return adj / det


E_PAD = 327680
_NW = 32
_EPW = E_PAD // _NW
_CHUNK = 1024
_NCHUNK = _EPW // _CHUNK
_DUMP = M_PAD * K_NB
_OUT_SZ = M_PAD * K_NB + 1024


def _sc_gather_scatter(posx, posy, posz, dst2d, slot2d):
    f32 = jnp.float32

    @functools.partial(
        pl.kernel,
        out_type=[jax.ShapeDtypeStruct((_OUT_SZ,), f32)] * 3,
        mesh=plsc.VectorSubcoreMesh(core_axis_name="c", subcore_axis_name="s"),
        scratch_types=[
            pltpu.VMEM((_CHUNK,), jnp.int32),
            pltpu.VMEM((_CHUNK,), jnp.int32),
            pltpu.VMEM((_CHUNK,), f32),
            pltpu.VMEM((_CHUNK,), f32),
            pltpu.VMEM((_CHUNK,), f32),
            pltpu.SemaphoreType.DMA,
        ],
    )
    def k(posx_h, posy_h, posz_h, dst_h, slot_h, ox_h, oy_h, oz_h,
          dstv, slotv, gx, gy, gz, sem):
        c = lax.axis_index("c")
        s = lax.axis_index("s")
        wid = s * 2 + c
        e0 = wid * _EPW

        def body(j, carry):
            base = e0 + j * _CHUNK
            pltpu.sync_copy(dst_h.at[pl.ds(base, _CHUNK)], dstv)
            pltpu.sync_copy(slot_h.at[pl.ds(base, _CHUNK)], slotv)
            pltpu.async_copy(posx_h.at[dstv], gx, sem).wait()
            pltpu.async_copy(posy_h.at[dstv], gy, sem).wait()
            pltpu.async_copy(posz_h.at[dstv], gz, sem).wait()
            pltpu.sync_copy(gx, ox_h.at[slotv])
            pltpu.sync_copy(gy, oy_h.at[slotv])
            pltpu.sync_copy(gz, oz_h.at[slotv])
            return carry

        lax.fori_loop(0, _NCHUNK, body, 0)

    return k(posx, posy, posz, dst2d, slot2d)


def _build_neighbors(positions, edge_index):
    src = edge_index[0]
    dst = edge_index[1]
    E = src.shape[0]
    order = jnp.argsort(src)
    src_s = src[order].astype(jnp.int32)
    dst_s = dst[order].astype(jnp.int32)
    iota = jnp.arange(E, dtype=jnp.int32)
    is_start = jnp.concatenate([jnp.ones((1,), bool), src_s[1:] != src_s[:-1]])
    first = lax.cummax(jnp.where(is_start, iota, 0))
    rank = iota - first
    keep = rank < K_NB
    slot = jnp.where(keep, rank * M_PAD + src_s, _DUMP)
    dst_pad = jnp.full((E_PAD,), 0, jnp.int32).at[:E].set(dst_s)
    slot_pad = jnp.full((E_PAD,), _DUMP, jnp.int32).at[:E].set(slot)
    ox, oy, oz = _sc_gather_scatter(
        positions[:, 0], positions[:, 1], positions[:, 2], dst_pad, slot_pad)
    nx = ox[:M_PAD * K_NB].reshape(K_NB, M_PAD)
    ny = oy[:M_PAD * K_NB].reshape(K_NB, M_PAD)
    nz = oz[:M_PAD * K_NB].reshape(K_NB, M_PAD)
    cnt = jnp.zeros((M_PAD,), jnp.int32).at[src].add(1)
    return nx, ny, nz, cnt


def kernel(positions, cell, edge_index, atom_indices):
    nx, ny, nz, cnt = _build_neighbors(positions, edge_index)
    cinv = _inv3(cell)
    cell = _bf_bits(cell)
    cinv = _bf_bits(cinv)
    cen_t = jnp.zeros((3, M_PAD), jnp.float32).at[:, :N_ATOMS_C].set(positions.T)
    cn, tet, q2 = _dense_call(cell, cinv, cen_t, cnt.reshape(1, M_PAD), nx, ny, nz)
    n = atom_indices.shape[0]
    return (cn[0, :n], tet[0, :n], q2[0, :n])


# speedup vs baseline: 1.6759x; 1.0167x over previous
import math
import functools

import jax
import jax.numpy as jnp
from jax import lax
from jax.experimental import pallas as pl
from jax.experimental.pallas import tpu as pltpu
from jax.experimental.pallas import tpu_sc as plsc

N_ATOMS_C = 10000
K_NB = 32
M_PAD = 10112
TILE = 128
N_TILES = M_PAD // TILE

_TET_TA = 0.6081734479693927 * math.pi
_TET_IDT = 1.0 / (12.0 * math.pi / 180.0 + 1e-10)
_C20 = 0.25 * math.sqrt(5.0 / math.pi)
_A21 = -0.5 * math.sqrt(15.0 / (2.0 * math.pi))
_A22 = 0.25 * math.sqrt(15.0 / (2.0 * math.pi))
_Q2S = 4.0 * math.pi / 5.0


def _acos(x):
    return jnp.arctan2(jnp.sqrt((1.0 - x) * (1.0 + x)), x)


def _bf(x):
    return x.astype(jnp.bfloat16).astype(jnp.float32)


def _bf_bits(x):
    u = jax.lax.bitcast_convert_type(x.astype(jnp.float32), jnp.uint32)
    bias = jnp.uint32(0x7FFF) + ((u >> 16) & jnp.uint32(1))
    u = (u + bias) & jnp.uint32(0xFFFF0000)
    return jax.lax.bitcast_convert_type(u, jnp.float32)


def _dense_body(cell_ref, cinv_ref, cen_ref, cnt_ref, nx_ref, ny_ref, nz_ref,
                cn_ref, tet_ref, q2_ref):
    K = K_NB
    f32 = jnp.float32
    krow = lax.broadcasted_iota(jnp.int32, (K, TILE), 0)
    cnt = cnt_ref[0:1, :]
    valid = krow < cnt
    vm = valid.astype(f32)

    cx = cen_ref[0:1, :]
    cy = cen_ref[1:2, :]
    cz = cen_ref[2:3, :]
    px = nx_ref[...]
    py = ny_ref[...]
    pz = nz_ref[...]
    px = jnp.where(valid, px, cx)
    py = jnp.where(valid, py, cy)
    pz = jnp.where(valid, pz, cz)
    dx = _bf(px - cx)
    dy = _bf(py - cy)
    dz = _bf(pz - cz)

    ci00 = cinv_ref[0, 0]; ci01 = cinv_ref[0, 1]; ci02 = cinv_ref[0, 2]
    ci10 = cinv_ref[1, 0]; ci11 = cinv_ref[1, 1]; ci12 = cinv_ref[1, 2]
    ci20 = cinv_ref[2, 0]; ci21 = cinv_ref[2, 1]; ci22 = cinv_ref[2, 2]
    c00 = cell_ref[0, 0]; c01 = cell_ref[0, 1]; c02 = cell_ref[0, 2]
    c10 = cell_ref[1, 0]; c11 = cell_ref[1, 1]; c12 = cell_ref[1, 2]
    c20 = cell_ref[2, 0]; c21 = cell_ref[2, 1]; c22 = cell_ref[2, 2]

    fx = dx * ci00 + dy * ci10 + dz * ci20
    fy = dx * ci01 + dy * ci11 + dz * ci21
    fz = dx * ci02 + dy * ci12 + dz * ci22
    fx = _bf(fx - jnp.round(fx))
    fy = _bf(fy - jnp.round(fy))
    fz = _bf(fz - jnp.round(fz))
    rx = fx * c00 + fy * c10 + fz * c20
    ry = fx * c01 + fy * c11 + fz * c21
    rz = fx * c02 + fy * c12 + fz * c22
    rx = (cx + rx) - cx
    ry = (cy + ry) - cy
    rz = (cz + rz) - cz

    d = jnp.sqrt(rx * rx + ry * ry + rz * rz + 1e-10)
    inv = 1.0 / (d + 1e-10)
    vx = rx * inv
    vy = ry * inv
    vz = rz * inv
    cos_t = jnp.clip(vz, -1.0 + 1e-7, 1.0 - 1e-7)
    th = _acos(cos_t)
    ph = jnp.arctan2(vy, vx)
    vx = jnp.where(valid, vx, 0.0)
    vy = jnp.where(valid, vy, 0.0)
    vz = jnp.where(valid, vz, 0.0)
    th = jnp.where(valid, th, 0.0)
    ph = jnp.where(valid, ph, 0.0)

    cnf = jnp.sum(vm, axis=0, keepdims=True)
    cn_ref[...] = cnf

    ct = jnp.cos(th)
    st = jnp.sin(th)
    cp = jnp.cos(ph)
    sp = jnp.sin(ph)
    nb = jnp.maximum(cnf, 1.0)
    inb = 1.0 / nb
    y20 = _C20 * (3.0 * ct * ct - 1.0)
    q20 = jnp.sum(y20 * vm, axis=0, keepdims=True) * inb
    a21 = _A21 * st * ct
    re21 = jnp.sum(a21 * cp * vm, axis=0, keepdims=True) * inb
    im21 = jnp.sum(a21 * sp * vm, axis=0, keepdims=True) * inb
    a22 = _A22 * st * st
    c2p = cp * cp - sp * sp
    s2p = 2.0 * sp * cp
    re22 = jnp.sum(a22 * c2p * vm, axis=0, keepdims=True) * inb
    im22 = jnp.sum(a22 * s2p * vm, axis=0, keepdims=True) * inb
    s = q20 * q20 + 2.0 * (re21 * re21 + im21 * im21) + 2.0 * (re22 * re22 + im22 * im22)
    q2_ref[...] = jnp.sqrt(_Q2S * s + 1e-12)

    gacc = jnp.zeros((K, TILE), f32)
    vxb = _bf(vx)
    vyb = _bf(vy)
    vzb = _bf(vz)
    for k in range(K):
        bx = vxb[k:k + 1, :]
        by = vyb[k:k + 1, :]
        bz = vzb[k:k + 1, :]
        vmk = vm[k:k + 1, :]
        dots = vxb * bx + vyb * by + vzb * bz
        dots = jnp.clip(dots, -1.0 + 1e-7, 1.0 - 1e-7)
        ang = _acos(dots)
        t = (ang - _TET_TA) * _TET_IDT
        ex = jnp.maximum(-0.5 * t * t, -50.0)
        g = jnp.exp(ex)
        fm = vm * vmk * (krow != k).astype(f32)
        gacc = gacc + g * fm
    gs = jnp.sum(gacc, axis=0, keepdims=True)
    npairs = cnf * (cnf - 1.0)
    tet_ref[...] = gs / jnp.maximum(npairs, 1.0)


def _dense_call(cell, cinv, cen_t, cnt, nx, ny, nz):
    out_shape = [jax.ShapeDtypeStruct((1, M_PAD), jnp.float32)] * 3
    grid = (N_TILES,)
    smem = functools.partial(pl.BlockSpec, memory_space=pltpu.SMEM)
    return pl.pallas_call(
        _dense_body,
        grid=grid,
        in_specs=[
            smem((3, 3), lambda i: (0, 0)),
            smem((3, 3), lambda i: (0, 0)),
            pl.BlockSpec((3, TILE), lambda i: (0, i)),
            pl.BlockSpec((1, TILE), lambda i: (0, i)),
            pl.BlockSpec((K_NB, TILE), lambda i: (0, i)),
            pl.BlockSpec((K_NB, TILE), lambda i: (0, i)),
            pl.BlockSpec((K_NB, TILE), lambda i: (0, i)),
        ],
        out_specs=[
            pl.BlockSpec((1, TILE), lambda i: (0, i)),
            pl.BlockSpec((1, TILE), lambda i: (0, i)),
            pl.BlockSpec((1, TILE), lambda i: (0, i)),
        ],
        out_shape=out_shape,
        compiler_params=pltpu.CompilerParams(
            dimension_semantics=("arbitrary",),
        ),
    )(cell, cinv, cen_t, cnt, nx, ny, nz)


def _inv3(m):
    a, b, c = m[0, 0], m[0, 1], m[0, 2]
    d, e, f = m[1, 0], m[1, 1], m[1, 2]
    g, h, i = m[2, 0], m[2, 1], m[2, 2]
    A = e * i - f * h
    B = f * g - d * i
    C = d * h - e * g
    det = a * A + b * B + c * C
    adj = jnp.stack([
        jnp.stack([A, c * h - b * i, b * f - c * e]),
        jnp.stack([B, a * i - c * g, c * d - a * f]),
        jnp.stack([C, b * g - a * h, a * e - b * d]),
    ])
    return adj / det


E_PAD = 327680
_NW = 32
_EPW = E_PAD // _NW
_CHUNK = 1024
_NCHUNK = _EPW // _CHUNK
_DUMP = M_PAD * K_NB
_OUT_SZ = M_PAD * K_NB + 1024


def _sc_gather_scatter(posx, posy, posz, dst_pad, slot_pad):
    f32 = jnp.float32
    i32 = jnp.int32

    @functools.partial(
        pl.kernel,
        out_type=[jax.ShapeDtypeStruct((_OUT_SZ,), f32)] * 3,
        mesh=plsc.VectorSubcoreMesh(core_axis_name="c", subcore_axis_name="s"),
        scratch_types=[
            pltpu.VMEM((_CHUNK,), i32), pltpu.VMEM((_CHUNK,), i32),
            pltpu.VMEM((_CHUNK,), i32), pltpu.VMEM((_CHUNK,), i32),
            pltpu.VMEM((_CHUNK,), f32), pltpu.VMEM((_CHUNK,), f32),
            pltpu.VMEM((_CHUNK,), f32), pltpu.VMEM((_CHUNK,), f32),
            pltpu.VMEM((_CHUNK,), f32), pltpu.VMEM((_CHUNK,), f32),
            pltpu.SemaphoreType.DMA, pltpu.SemaphoreType.DMA,
            pltpu.SemaphoreType.DMA, pltpu.SemaphoreType.DMA,
        ],
    )
    def k(posx_h, posy_h, posz_h, dst_h, slot_h, ox_h, oy_h, oz_h,
          dstv0, dstv1, slotv0, slotv1, gx0, gx1, gy0, gy1, gz0, gz1,
          semg0, semg1, sems0, sems1):
        c = lax.axis_index("c")
        s = lax.axis_index("s")
        wid = s * 2 + c
        e0 = wid * _EPW
        dstv = (dstv0, dstv1)
        slotv = (slotv0, slotv1)
        gx = (gx0, gx1)
        gy = (gy0, gy1)
        gz = (gz0, gz1)
        semg = (semg0, semg1)
        sems = (sems0, sems1)
        pend = [None, None]
        for j in range(_NCHUNK):
            b = j % 2
            base = e0 + j * _CHUNK
            if pend[b] is not None:
                for h in pend[b]:
                    h.wait()
            pltpu.sync_copy(dst_h.at[pl.ds(base, _CHUNK)], dstv[b])
            pltpu.sync_copy(slot_h.at[pl.ds(base, _CHUNK)], slotv[b])
            h1 = pltpu.async_copy(posx_h.at[dstv[b]], gx[b], semg[b])
            h2 = pltpu.async_copy(posy_h.at[dstv[b]], gy[b], semg[b])
            h3 = pltpu.async_copy(posz_h.at[dstv[b]], gz[b], semg[b])
            h1.wait(); h2.wait(); h3.wait()
            pend[b] = (
                pltpu.async_copy(gx[b], ox_h.at[slotv[b]], sems[b]),
                pltpu.async_copy(gy[b], oy_h.at[slotv[b]], sems[b]),
                pltpu.async_copy(gz[b], oz_h.at[slotv[b]], sems[b]),
            )
        for b in range(2):
            if pend[b] is not None:
                for h in pend[b]:
                    h.wait()

    return k(posx, posy, posz, dst_pad, slot_pad)


def _build_neighbors(positions, edge_index):
    src = edge_index[0]
    dst = edge_index[1]
    E = src.shape[0]
    order = jnp.argsort(src)
    src_s = src[order].astype(jnp.int32)
    dst_s = dst[order].astype(jnp.int32)
    iota = jnp.arange(E, dtype=jnp.int32)
    is_start = jnp.concatenate([jnp.ones((1,), bool), src_s[1:] != src_s[:-1]])
    first = lax.cummax(jnp.where(is_start, iota, 0))
    rank = iota - first
    keep = rank < K_NB
    slot = jnp.where(keep, rank * M_PAD + src_s, _DUMP)
    dst_pad = jnp.full((E_PAD,), 0, jnp.int32).at[:E].set(dst_s)
    slot_pad = jnp.full((E_PAD,), _DUMP, jnp.int32).at[:E].set(slot)
    ox, oy, oz = _sc_gather_scatter(
        positions[:, 0], positions[:, 1], positions[:, 2], dst_pad, slot_pad)
    nx = ox[:M_PAD * K_NB].reshape(K_NB, M_PAD)
    ny = oy[:M_PAD * K_NB].reshape(K_NB, M_PAD)
    nz = oz[:M_PAD * K_NB].reshape(K_NB, M_PAD)
    cnt = jnp.zeros((M_PAD,), jnp.int32).at[src].add(1)
    return nx, ny, nz, cnt


def kernel(positions, cell, edge_index, atom_indices):
    nx, ny, nz, cnt = _build_neighbors(positions, edge_index)
    cinv = _inv3(cell)
    cell = _bf_bits(cell)
    cinv = _bf_bits(cinv)
    cen_t = jnp.zeros((3, M_PAD), jnp.float32).at[:, :N_ATOMS_C].set(positions.T)
    cn, tet, q2 = _dense_call(cell, cinv, cen_t, cnt.reshape(1, M_PAD), nx, ny, nz)
    n = atom_indices.shape[0]
    return (cn[0, :n], tet[0, :n], q2[0, :n])
